# baseline (speedup 1.0000x reference)
import jax
import jax.numpy as jnp
from jax.experimental import pallas as pl
from jax.experimental.pallas import tpu as pltpu
from jax.experimental.pallas import tpu_sc as plsc

_S = 128


def kernel(x, table):
    B, L = x.shape
    V, E = table.shape
    idx = x.astype(jnp.int32)
    mesh = plsc.VectorSubcoreMesh(core_axis_name="core", subcore_axis_name="subcore")

    @pl.kernel(
        out_type=jax.ShapeDtypeStruct((B, L, E), table.dtype),
        mesh=mesh,
        compiler_params=pltpu.CompilerParams(use_tc_tiling_on_sc=False),
    )
    def _gather(tab_hbm, i_hbm, o_hbm):
        def body(i_vmem, o_vmem):
            for lo in range(0, L, _S):
                w = min(_S, L - lo)
                pltpu.sync_copy(
                    tab_hbm.at[i_vmem.at[0, pl.ds(lo, w)]],
                    o_vmem.at[0, pl.ds(lo, w), :],
                )

        pltpu.emit_pipeline(
            body,
            grid=(B,),
            in_specs=[pl.BlockSpec((1, L), index_map=lambda i: (i, 0))],
            out_specs=[pl.BlockSpec((1, L, E), index_map=lambda i: (i, 0, 0))],
            core_axis_name=("core", "subcore"),
            dimension_semantics=(pltpu.PARALLEL,),
        )(i_hbm, o_hbm)

    return _gather(table, idx)
